# baseline (speedup 1.0000x reference)
import functools

import jax
import jax.numpy as jnp
from jax import lax
from jax.experimental import pallas as pl
from jax.experimental.pallas import tpu as pltpu
from jax.experimental.pallas import tpu_sc as plsc

N = 10000
E = 320000
D = 128
NC = 2
NS = 16
CH = 80
NQ = 5
NJ = 25
NCHUNK = NQ * NJ
N_PAD = 10240
FROWS = N_PAD // NS
LROWS = N - (NS - 1) * FROWS
ZCH = 40
DEGW = 8

_mesh = plsc.VectorSubcoreMesh(
    core_axis_name="c", subcore_axis_name="s", num_cores=NC, num_subcores=NS)


@functools.partial(
    pl.kernel,
    out_type=jax.ShapeDtypeStruct((6, NC, N_PAD), jnp.float32),
    mesh=_mesh,
    scratch_types=[
        pltpu.VMEM((NJ, CH), jnp.int32),
        pltpu.VMEM((N_PAD,), jnp.float32),
        pltpu.VMEM((N_PAD,), jnp.float32),
        pltpu.VMEM((FROWS,), jnp.float32),
        pltpu.VMEM((FROWS,), jnp.float32),
        pltpu.VMEM_SHARED((NS, N_PAD), jnp.float32),
    ],
    compiler_params=pltpu.CompilerParams(needs_layout_passes=False),
)
def _sc_degrees(idx_hbm, zeros_hbm, deg_hbm,
                idx_v, hist_v, hist2_v, acc_v, tmp_v, sh):
    c = lax.axis_index("c")
    s = lax.axis_index("s")
    zeros16 = jnp.zeros((16,), jnp.float32)
    ones16 = jnp.ones((16,), jnp.float32)
    lane_base = pl.multiple_of(s * FROWS, 8)
    for k in range(6):
        pltpu.sync_copy(zeros_hbm, hist_v)
        pltpu.sync_copy(zeros_hbm, hist2_v)

        def qbody(q, carry, k=k):
            pltpu.sync_copy(idx_hbm.at[k, c, s, q], idx_v)

            def jbody(j, carry2):
                vs = [idx_v[j, pl.ds(t * 16, 16)] for t in range(CH // 16)]
                for t, v in enumerate(vs):
                    plsc.addupdate_scatter(
                        hist_v if t % 2 == 0 else hist2_v, [v], ones16)
                return carry2

            return lax.fori_loop(0, NJ, jbody, carry)

        lax.fori_loop(0, NQ, qbody, 0)

        def mb(i, carry):
            sl = pl.ds(i * 16, 16)
            hist_v[sl] = hist_v[sl] + hist2_v[sl]
            return carry

        lax.fori_loop(0, N_PAD // 16, mb, 0)
        pltpu.sync_copy(hist_v, sh.at[s])
        plsc.subcore_barrier()

        def zb2(i, carry):
            acc_v[pl.ds(i * 16, 16)] = zeros16
            return carry

        lax.fori_loop(0, FROWS // 16, zb2, 0)
        for i in range(NS):
            pltpu.sync_copy(sh.at[i, pl.ds(lane_base, FROWS)], tmp_v)

            def ab(m, carry):
                sl = pl.ds(m * 16, 16)
                acc_v[sl] = acc_v[sl] + tmp_v[sl]
                return carry

            lax.fori_loop(0, FROWS // 16, ab, 0)
        pltpu.sync_copy(acc_v, deg_hbm.at[k, c, pl.ds(lane_base, FROWS)])
        plsc.subcore_barrier()


_PART_T = jax.ShapeDtypeStruct((NC, N, D), jnp.float32)


@functools.partial(
    pl.kernel,
    out_type=(_PART_T, _PART_T, _PART_T),
    mesh=_mesh,
    scratch_types=[
        pltpu.VMEM((NJ, CH), jnp.int32),
        pltpu.VMEM((NJ, CH), jnp.int32),
        pltpu.VMEM((CH, D), jnp.float32),
        pltpu.VMEM((CH, D), jnp.float32),
        pltpu.VMEM((ZCH, D), jnp.float32),
        pltpu.VMEM_SHARED((N_PAD, D), jnp.float32),
        pltpu.SemaphoreType.DMA,
        pltpu.SemaphoreType.DMA,
        pltpu.SemaphoreType.DMA,
        pltpu.SemaphoreType.DMA,
    ],
)
def _sc_aggregate(idx_hbm, g0_hbm, g1_hbm, g2_hbm, zeros_hbm,
                  p0_hbm, p1_hbm, p2_hbm,
                  src_v, dst_v, rows_a, rows_b, zero_v, agg_sh,
                  sem_ga, sem_gb, sem_sa, sem_sb):
    c = lax.axis_index("c")
    s = lax.axis_index("s")
    base = pl.multiple_of(s * FROWS, 8)
    n_zero = jnp.where(s < NS - 1, FROWS // ZCH, LROWS // ZCH)
    pltpu.sync_copy(zeros_hbm, zero_v)
    gs = (g0_hbm, g1_hbm, g2_hbm)
    ps = (p0_hbm, p1_hbm, p2_hbm)
    for r in range(3):
        g = gs[r]

        def zbody(z, carry):
            off = pl.multiple_of(base + z * ZCH, 8)
            pltpu.sync_copy(zero_v, agg_sh.at[pl.ds(off, ZCH), :])
            return carry

        lax.fori_loop(0, n_zero, zbody, 0)
        plsc.subcore_barrier()

        def qblock(q, carry, g=g, r=r):
            pltpu.sync_copy(idx_hbm.at[2 * r, c, s, q], src_v)
            pltpu.sync_copy(idx_hbm.at[2 * r + 1, c, s, q], dst_v)

            def startg(w, buf, sem):
                pltpu.async_copy(g.at[src_v.at[w]], buf, sem)

            def waitg(w, buf, sem):
                pltpu.make_async_copy(g.at[src_v.at[w]], buf, sem).wait()

            def starts(w, buf, sem):
                pltpu.async_copy(buf, agg_sh.at[dst_v.at[w]], sem, add=True)

            def waits(w, buf, sem):
                pltpu.make_async_copy(buf, agg_sh.at[dst_v.at[w]], sem).wait()

            startg(0, rows_a, sem_ga)
            startg(1, rows_b, sem_gb)
            waitg(0, rows_a, sem_ga)
            starts(0, rows_a, sem_sa)

            def pair(m, carry2):
                w = 2 * m
                waits(w - 2, rows_a, sem_sa)
                startg(w, rows_a, sem_ga)
                waitg(w - 1, rows_b, sem_gb)
                starts(w - 1, rows_b, sem_sb)
                waits(w - 1, rows_b, sem_sb)
                startg(w + 1, rows_b, sem_gb)
                waitg(w, rows_a, sem_ga)
                starts(w, rows_a, sem_sa)
                return carry2

            lax.fori_loop(1, (NJ - 1) // 2, pair, 0)
            waits(NJ - 3, rows_a, sem_sa)
            startg(NJ - 1, rows_a, sem_ga)
            waitg(NJ - 2, rows_b, sem_gb)
            starts(NJ - 2, rows_b, sem_sb)
            waitg(NJ - 1, rows_a, sem_ga)
            starts(NJ - 1, rows_a, sem_sa)
            waits(NJ - 2, rows_b, sem_sb)
            waits(NJ - 1, rows_a, sem_sa)
            return carry

        lax.fori_loop(0, NQ, qblock, 0)
        plsc.subcore_barrier()

        @pl.when(s < NS - 1)
        def _(r=r):
            pltpu.sync_copy(agg_sh.at[pl.ds(base, FROWS), :],
                            ps[r].at[c, pl.ds(base, FROWS), :])

        @pl.when(s == NS - 1)
        def _(r=r):
            pltpu.sync_copy(agg_sh.at[pl.ds(base, LROWS), :],
                            ps[r].at[c, pl.ds(base, LROWS), :])

        plsc.subcore_barrier()


R_BLK = 1000
GRID = N // R_BLK


def _tc_prep_body(x_ref, deg_ref, norms_ref, g0_ref, g1_ref, g2_ref):
    x = x_ref[...]
    dd = deg_ref[...]
    cols = []
    for k in range(6):
        d = dd[k, 0] + dd[k, 1]
        cols.append(jnp.where(d > 0, lax.rsqrt(jnp.maximum(d, 1e-30)), 0.0))
    pad = jnp.zeros_like(cols[0])
    norms_ref[...] = jnp.concatenate(cols + [pad, pad], axis=1)
    g_refs = (g0_ref, g1_ref, g2_ref)
    for r in range(3):
        g_refs[r][...] = x * cols[2 * r]


def _tc_prep(x, degp):
    return pl.pallas_call(
        _tc_prep_body,
        grid=(GRID,),
        in_specs=[
            pl.BlockSpec((R_BLK, D), lambda i: (i, 0)),
            pl.BlockSpec((6, NC, R_BLK, 1), lambda i: (0, 0, i, 0)),
        ],
        out_specs=[
            pl.BlockSpec((R_BLK, DEGW), lambda i: (i, 0)),
            pl.BlockSpec((R_BLK, D), lambda i: (i, 0)),
            pl.BlockSpec((R_BLK, D), lambda i: (i, 0)),
            pl.BlockSpec((R_BLK, D), lambda i: (i, 0)),
        ],
        out_shape=[
            jax.ShapeDtypeStruct((N, DEGW), jnp.float32),
            jax.ShapeDtypeStruct((N, D), jnp.float32),
            jax.ShapeDtypeStruct((N, D), jnp.float32),
            jax.ShapeDtypeStruct((N, D), jnp.float32),
        ],
    )(x, degp)


def _tc_layer_body(p0_ref, p1_ref, p2_ref, norms_ref, w_ref, b_ref, *out_refs,
                   rescale):
    norms = norms_ref[...]
    mats = []
    for r in range(3):
        pr = (p0_ref, p1_ref, p2_ref)[r][...]
        agg = pr[0] + pr[1]
        mats.append(agg * norms[:, 2 * r + 1:2 * r + 2])
    cat = jnp.concatenate(mats, axis=1)
    t = jnp.dot(cat, w_ref[...], preferred_element_type=jnp.float32)
    t = t + b_ref[...]
    if rescale:
        h = jnp.maximum(t, 0.0)
        for r in range(3):
            out_refs[r][...] = h * norms[:, 2 * r:2 * r + 1]
    else:
        out_refs[0][...] = t


def _tc_layer(p0, p1, p2, norms, wcat, bsum, rescale):
    n_out = 3 if rescale else 1
    return pl.pallas_call(
        functools.partial(_tc_layer_body, rescale=rescale),
        grid=(GRID,),
        in_specs=[
            pl.BlockSpec((NC, R_BLK, D), lambda i: (0, i, 0)),
            pl.BlockSpec((NC, R_BLK, D), lambda i: (0, i, 0)),
            pl.BlockSpec((NC, R_BLK, D), lambda i: (0, i, 0)),
            pl.BlockSpec((R_BLK, DEGW), lambda i: (i, 0)),
            pl.BlockSpec((3 * D, D), lambda i: (0, 0)),
            pl.BlockSpec((1, D), lambda i: (0, 0)),
        ],
        out_specs=[pl.BlockSpec((R_BLK, D), lambda i: (i, 0))] * n_out,
        out_shape=[jax.ShapeDtypeStruct((N, D), jnp.float32)] * n_out,
    )(p0, p1, p2, norms, wcat, bsum)


def kernel(x, edge_index_b, edge_index_r, edge_index_j,
           W1_b, W1_r, W1_j, b1_b, b1_r, b1_j,
           W2_b, W2_r, W2_j, b2_b, b2_r, b2_j):
    idx_stack = jnp.stack([
        edge_index_b[0], edge_index_b[1],
        edge_index_r[0], edge_index_r[1],
        edge_index_j[0], edge_index_j[1],
    ]).astype(jnp.int32)
    idx_all = idx_stack.reshape(6, NC, NS, NQ, NJ, CH)

    zeros_agg = jnp.zeros((ZCH, D), jnp.float32)
    zeros_hist = jnp.zeros((N_PAD,), jnp.float32)

    degp = _sc_degrees(idx_all, zeros_hist)
    degt = degp[:, :, :N, None]
    norms, g10, g11, g12 = _tc_prep(x, degt)

    p10, p11, p12 = _sc_aggregate(idx_all, g10, g11, g12, zeros_agg)
    w1cat = jnp.concatenate([W1_b, W1_r, W1_j], axis=0)
    b1sum = (b1_b + b1_r + b1_j)[None, :]
    g20, g21, g22 = _tc_layer(p10, p11, p12, norms, w1cat, b1sum, True)

    p20, p21, p22 = _sc_aggregate(idx_all, g20, g21, g22, zeros_agg)
    w2cat = jnp.concatenate([W2_b, W2_r, W2_j], axis=0)
    b2sum = (b2_b + b2_r + b2_j)[None, :]
    (out,) = _tc_layer(p20, p21, p22, norms, w2cat, b2sum, False)
    return out


# speedup vs baseline: 1.0157x; 1.0157x over previous
import functools

import jax
import jax.numpy as jnp
from jax import lax
from jax.experimental import pallas as pl
from jax.experimental.pallas import tpu as pltpu
from jax.experimental.pallas import tpu_sc as plsc

N = 10000
E = 320000
D = 128
NC = 2
NS = 16
CH = 80
NQ = 5
NJ = 25
NCHUNK = NQ * NJ
N_PAD = 10240
FROWS = N_PAD // NS
LROWS = N - (NS - 1) * FROWS
ZCH = 40
DEGW = 8

_mesh = plsc.VectorSubcoreMesh(
    core_axis_name="c", subcore_axis_name="s", num_cores=NC, num_subcores=NS)


@functools.partial(
    pl.kernel,
    out_type=jax.ShapeDtypeStruct((6, NC, N_PAD), jnp.float32),
    mesh=_mesh,
    scratch_types=[
        pltpu.VMEM((NJ, CH), jnp.int32),
        pltpu.VMEM((N_PAD,), jnp.float32),
        pltpu.VMEM((FROWS,), jnp.float32),
        pltpu.VMEM((FROWS,), jnp.float32),
        pltpu.VMEM_SHARED((NS, N_PAD), jnp.float32),
    ],
    compiler_params=pltpu.CompilerParams(needs_layout_passes=False),
)
def _sc_degrees(idx_hbm, zeros_hbm, deg_hbm,
                idx_v, hist_v, acc_v, tmp_v, sh):
    c = lax.axis_index("c")
    s = lax.axis_index("s")
    zeros16 = jnp.zeros((16,), jnp.float32)
    ones16 = jnp.ones((16,), jnp.float32)
    lane_base = pl.multiple_of(s * FROWS, 8)
    for k in range(6):
        pltpu.sync_copy(zeros_hbm, hist_v)

        def qbody(q, carry, k=k):
            pltpu.sync_copy(idx_hbm.at[k, c, s, q], idx_v)

            def jbody(j, carry2):
                for t in range(CH // 16):
                    v = idx_v[j, pl.ds(t * 16, 16)]
                    plsc.addupdate_scatter(hist_v, [v], ones16)
                return carry2

            return lax.fori_loop(0, NJ, jbody, carry)

        lax.fori_loop(0, NQ, qbody, 0)
        pltpu.sync_copy(hist_v, sh.at[s])
        plsc.subcore_barrier()

        def zb2(i, carry):
            acc_v[pl.ds(i * 16, 16)] = zeros16
            return carry

        lax.fori_loop(0, FROWS // 16, zb2, 0)
        for i in range(NS):
            pltpu.sync_copy(sh.at[i, pl.ds(lane_base, FROWS)], tmp_v)

            def ab(m, carry):
                sl = pl.ds(m * 16, 16)
                acc_v[sl] = acc_v[sl] + tmp_v[sl]
                return carry

            lax.fori_loop(0, FROWS // 16, ab, 0)
        pltpu.sync_copy(acc_v, deg_hbm.at[k, c, pl.ds(lane_base, FROWS)])
        plsc.subcore_barrier()


_PART_T = jax.ShapeDtypeStruct((NC, N, D), jnp.float32)


@functools.partial(
    pl.kernel,
    out_type=(_PART_T, _PART_T, _PART_T),
    mesh=_mesh,
    scratch_types=[
        pltpu.VMEM((NJ, CH), jnp.int32),
        pltpu.VMEM((NJ, CH), jnp.int32),
        pltpu.VMEM((CH, D), jnp.float32),
        pltpu.VMEM((CH, D), jnp.float32),
        pltpu.VMEM((ZCH, D), jnp.float32),
        pltpu.VMEM_SHARED((N_PAD, D), jnp.float32),
        pltpu.SemaphoreType.DMA,
        pltpu.SemaphoreType.DMA,
        pltpu.SemaphoreType.DMA,
        pltpu.SemaphoreType.DMA,
    ],
)
def _sc_aggregate(idx_hbm, g0_hbm, g1_hbm, g2_hbm, zeros_hbm,
                  p0_hbm, p1_hbm, p2_hbm,
                  src_v, dst_v, rows_a, rows_b, zero_v, agg_sh,
                  sem_ga, sem_gb, sem_sa, sem_sb):
    c = lax.axis_index("c")
    s = lax.axis_index("s")
    base = pl.multiple_of(s * FROWS, 8)
    n_zero = jnp.where(s < NS - 1, FROWS // ZCH, LROWS // ZCH)
    pltpu.sync_copy(zeros_hbm, zero_v)
    gs = (g0_hbm, g1_hbm, g2_hbm)
    ps = (p0_hbm, p1_hbm, p2_hbm)
    for r in range(3):
        g = gs[r]

        def zbody(z, carry):
            off = pl.multiple_of(base + z * ZCH, 8)
            pltpu.sync_copy(zero_v, agg_sh.at[pl.ds(off, ZCH), :])
            return carry

        lax.fori_loop(0, n_zero, zbody, 0)
        plsc.subcore_barrier()

        def qblock(q, carry, g=g, r=r):
            pltpu.sync_copy(idx_hbm.at[2 * r, c, s, q], src_v)
            pltpu.sync_copy(idx_hbm.at[2 * r + 1, c, s, q], dst_v)

            def startg(w, buf, sem):
                pltpu.async_copy(g.at[src_v.at[w]], buf, sem)

            def waitg(w, buf, sem):
                pltpu.make_async_copy(g.at[src_v.at[w]], buf, sem).wait()

            def starts(w, buf, sem):
                pltpu.async_copy(buf, agg_sh.at[dst_v.at[w]], sem, add=True)

            def waits(w, buf, sem):
                pltpu.make_async_copy(buf, agg_sh.at[dst_v.at[w]], sem).wait()

            startg(0, rows_a, sem_ga)
            startg(1, rows_b, sem_gb)
            waitg(0, rows_a, sem_ga)
            starts(0, rows_a, sem_sa)

            def pair(m, carry2):
                w = 2 * m
                waits(w - 2, rows_a, sem_sa)
                startg(w, rows_a, sem_ga)
                waitg(w - 1, rows_b, sem_gb)
                starts(w - 1, rows_b, sem_sb)
                waits(w - 1, rows_b, sem_sb)
                startg(w + 1, rows_b, sem_gb)
                waitg(w, rows_a, sem_ga)
                starts(w, rows_a, sem_sa)
                return carry2

            lax.fori_loop(1, (NJ - 1) // 2, pair, 0)
            waits(NJ - 3, rows_a, sem_sa)
            startg(NJ - 1, rows_a, sem_ga)
            waitg(NJ - 2, rows_b, sem_gb)
            starts(NJ - 2, rows_b, sem_sb)
            waitg(NJ - 1, rows_a, sem_ga)
            starts(NJ - 1, rows_a, sem_sa)
            waits(NJ - 2, rows_b, sem_sb)
            waits(NJ - 1, rows_a, sem_sa)
            return carry

        lax.fori_loop(0, NQ, qblock, 0)
        plsc.subcore_barrier()

        @pl.when(s < NS - 1)
        def _(r=r):
            pltpu.sync_copy(agg_sh.at[pl.ds(base, FROWS), :],
                            ps[r].at[c, pl.ds(base, FROWS), :])

        @pl.when(s == NS - 1)
        def _(r=r):
            pltpu.sync_copy(agg_sh.at[pl.ds(base, LROWS), :],
                            ps[r].at[c, pl.ds(base, LROWS), :])

        plsc.subcore_barrier()


R_BLK = 1000
GRID = N // R_BLK


def _tc_prep_body(x_ref, deg_ref, norms_ref, g0_ref, g1_ref, g2_ref):
    x = x_ref[...]
    dd = deg_ref[...]
    cols = []
    for k in range(6):
        d = dd[k, 0] + dd[k, 1]
        cols.append(jnp.where(d > 0, lax.rsqrt(jnp.maximum(d, 1e-30)), 0.0))
    pad = jnp.zeros_like(cols[0])
    norms_ref[...] = jnp.concatenate(cols + [pad, pad], axis=1)
    g_refs = (g0_ref, g1_ref, g2_ref)
    for r in range(3):
        g_refs[r][...] = x * cols[2 * r]


def _tc_prep(x, degp):
    return pl.pallas_call(
        _tc_prep_body,
        grid=(GRID,),
        in_specs=[
            pl.BlockSpec((R_BLK, D), lambda i: (i, 0)),
            pl.BlockSpec((6, NC, R_BLK, 1), lambda i: (0, 0, i, 0)),
        ],
        out_specs=[
            pl.BlockSpec((R_BLK, DEGW), lambda i: (i, 0)),
            pl.BlockSpec((R_BLK, D), lambda i: (i, 0)),
            pl.BlockSpec((R_BLK, D), lambda i: (i, 0)),
            pl.BlockSpec((R_BLK, D), lambda i: (i, 0)),
        ],
        out_shape=[
            jax.ShapeDtypeStruct((N, DEGW), jnp.float32),
            jax.ShapeDtypeStruct((N, D), jnp.float32),
            jax.ShapeDtypeStruct((N, D), jnp.float32),
            jax.ShapeDtypeStruct((N, D), jnp.float32),
        ],
    )(x, degp)


def _tc_layer_body(p0_ref, p1_ref, p2_ref, norms_ref, w_ref, b_ref, *out_refs,
                   rescale):
    norms = norms_ref[...]
    mats = []
    for r in range(3):
        pr = (p0_ref, p1_ref, p2_ref)[r][...]
        agg = pr[0] + pr[1]
        mats.append(agg * norms[:, 2 * r + 1:2 * r + 2])
    cat = jnp.concatenate(mats, axis=1)
    t = jnp.dot(cat, w_ref[...], preferred_element_type=jnp.float32)
    t = t + b_ref[...]
    if rescale:
        h = jnp.maximum(t, 0.0)
        for r in range(3):
            out_refs[r][...] = h * norms[:, 2 * r:2 * r + 1]
    else:
        out_refs[0][...] = t


def _tc_layer(p0, p1, p2, norms, wcat, bsum, rescale):
    n_out = 3 if rescale else 1
    return pl.pallas_call(
        functools.partial(_tc_layer_body, rescale=rescale),
        grid=(GRID,),
        in_specs=[
            pl.BlockSpec((NC, R_BLK, D), lambda i: (0, i, 0)),
            pl.BlockSpec((NC, R_BLK, D), lambda i: (0, i, 0)),
            pl.BlockSpec((NC, R_BLK, D), lambda i: (0, i, 0)),
            pl.BlockSpec((R_BLK, DEGW), lambda i: (i, 0)),
            pl.BlockSpec((3 * D, D), lambda i: (0, 0)),
            pl.BlockSpec((1, D), lambda i: (0, 0)),
        ],
        out_specs=[pl.BlockSpec((R_BLK, D), lambda i: (i, 0))] * n_out,
        out_shape=[jax.ShapeDtypeStruct((N, D), jnp.float32)] * n_out,
    )(p0, p1, p2, norms, wcat, bsum)


def kernel(x, edge_index_b, edge_index_r, edge_index_j,
           W1_b, W1_r, W1_j, b1_b, b1_r, b1_j,
           W2_b, W2_r, W2_j, b2_b, b2_r, b2_j):
    idx_stack = jnp.stack([
        edge_index_b[0], edge_index_b[1],
        edge_index_r[0], edge_index_r[1],
        edge_index_j[0], edge_index_j[1],
    ]).astype(jnp.int32)
    idx_all = idx_stack.reshape(6, NC, NS, NQ, NJ, CH)

    zeros_agg = jnp.zeros((ZCH, D), jnp.float32)
    zeros_hist = jnp.zeros((N_PAD,), jnp.float32)

    degp = _sc_degrees(idx_all, zeros_hist)
    degt = degp[:, :, :N, None]
    norms, g10, g11, g12 = _tc_prep(x, degt)

    p10, p11, p12 = _sc_aggregate(idx_all, g10, g11, g12, zeros_agg)
    w1cat = jnp.concatenate([W1_b, W1_r, W1_j], axis=0)
    b1sum = (b1_b + b1_r + b1_j)[None, :]
    g20, g21, g22 = _tc_layer(p10, p11, p12, norms, w1cat, b1sum, True)

    p20, p21, p22 = _sc_aggregate(idx_all, g20, g21, g22, zeros_agg)
    w2cat = jnp.concatenate([W2_b, W2_r, W2_j], axis=0)
    b2sum = (b2_b + b2_r + b2_j)[None, :]
    (out,) = _tc_layer(p20, p21, p22, norms, w2cat, b2sum, False)
    return out


# speedup vs baseline: 1.0623x; 1.0459x over previous
import functools

import jax
import jax.numpy as jnp
from jax import lax
from jax.experimental import pallas as pl
from jax.experimental.pallas import tpu as pltpu
from jax.experimental.pallas import tpu_sc as plsc

N = 10000
E = 320000
D = 128
NC = 2
NS = 16
CH = 80
NQ = 5
NJ = 25
NCHUNK = NQ * NJ
N_PAD = 10240
FROWS = N_PAD // NS
LROWS = N - (NS - 1) * FROWS
ZCH = 40
DEGW = 8

_mesh = plsc.VectorSubcoreMesh(
    core_axis_name="c", subcore_axis_name="s", num_cores=NC, num_subcores=NS)


@functools.partial(
    pl.kernel,
    out_type=jax.ShapeDtypeStruct((6, NC, N_PAD), jnp.float32),
    mesh=_mesh,
    scratch_types=[
        pltpu.VMEM((NJ, CH), jnp.int32),
        pltpu.VMEM((NJ, CH), jnp.int32),
        pltpu.VMEM((N_PAD,), jnp.float32),
        pltpu.VMEM((FROWS,), jnp.float32),
        pltpu.VMEM((FROWS,), jnp.float32),
        pltpu.VMEM_SHARED((NS, N_PAD), jnp.float32),
        pltpu.SemaphoreType.DMA,
    ],
    compiler_params=pltpu.CompilerParams(needs_layout_passes=False),
)
def _sc_degrees(idx_hbm, zeros_hbm, deg_hbm,
                idx_v, idx_v2, hist_v, acc_v, tmp_v, sh, sem_i):
    c = lax.axis_index("c")
    s = lax.axis_index("s")
    zeros16 = jnp.zeros((16,), jnp.float32)
    ones16 = jnp.ones((16,), jnp.float32)
    lane_base = pl.multiple_of(s * FROWS, 8)
    ibufs = (idx_v, idx_v2)
    for k in range(6):
        pltpu.sync_copy(zeros_hbm, hist_v)
        pltpu.sync_copy(idx_hbm.at[k, c, s, 0], idx_v)
        for q in range(NQ):
            iv = ibufs[q % 2]
            if q + 1 < NQ:
                pltpu.async_copy(
                    idx_hbm.at[k, c, s, q + 1], ibufs[(q + 1) % 2], sem_i)

            def jbody(j, carry2, iv=iv):
                for t in range(CH // 16):
                    v = iv[j, pl.ds(t * 16, 16)]
                    plsc.addupdate_scatter(hist_v, [v], ones16)
                return carry2

            lax.fori_loop(0, NJ, jbody, 0)
            if q + 1 < NQ:
                pltpu.make_async_copy(
                    idx_hbm.at[k, c, s, q + 1], ibufs[(q + 1) % 2],
                    sem_i).wait()
        pltpu.sync_copy(hist_v, sh.at[s])
        plsc.subcore_barrier()

        def zb2(i, carry):
            acc_v[pl.ds(i * 16, 16)] = zeros16
            return carry

        lax.fori_loop(0, FROWS // 16, zb2, 0)
        for i in range(NS):
            pltpu.sync_copy(sh.at[i, pl.ds(lane_base, FROWS)], tmp_v)

            def ab(m, carry):
                sl = pl.ds(m * 16, 16)
                acc_v[sl] = acc_v[sl] + tmp_v[sl]
                return carry

            lax.fori_loop(0, FROWS // 16, ab, 0)
        pltpu.sync_copy(acc_v, deg_hbm.at[k, c, pl.ds(lane_base, FROWS)])
        plsc.subcore_barrier()


_PART_T = jax.ShapeDtypeStruct((NC, N, D), jnp.float32)


@functools.partial(
    pl.kernel,
    out_type=(_PART_T, _PART_T, _PART_T),
    mesh=_mesh,
    scratch_types=[
        pltpu.VMEM((NJ, CH), jnp.int32),
        pltpu.VMEM((NJ, CH), jnp.int32),
        pltpu.VMEM((NJ, CH), jnp.int32),
        pltpu.VMEM((NJ, CH), jnp.int32),
        pltpu.VMEM((CH, D), jnp.float32),
        pltpu.VMEM((CH, D), jnp.float32),
        pltpu.VMEM((ZCH, D), jnp.float32),
        pltpu.VMEM_SHARED((N_PAD, D), jnp.float32),
        pltpu.SemaphoreType.DMA,
        pltpu.SemaphoreType.DMA,
        pltpu.SemaphoreType.DMA,
        pltpu.SemaphoreType.DMA,
        pltpu.SemaphoreType.DMA,
    ],
)
def _sc_aggregate(idx_hbm, g0_hbm, g1_hbm, g2_hbm, zeros_hbm,
                  p0_hbm, p1_hbm, p2_hbm,
                  src_v, src_v2, dst_v, dst_v2, rows_a, rows_b, zero_v, agg_sh,
                  sem_ga, sem_gb, sem_sa, sem_sb, sem_i):
    c = lax.axis_index("c")
    s = lax.axis_index("s")
    base = pl.multiple_of(s * FROWS, 8)
    n_zero = jnp.where(s < NS - 1, FROWS // ZCH, LROWS // ZCH)
    pltpu.sync_copy(zeros_hbm, zero_v)
    gs = (g0_hbm, g1_hbm, g2_hbm)
    ps = (p0_hbm, p1_hbm, p2_hbm)
    for r in range(3):
        g = gs[r]

        def zbody(z, carry):
            off = pl.multiple_of(base + z * ZCH, 8)
            pltpu.sync_copy(zero_v, agg_sh.at[pl.ds(off, ZCH), :])
            return carry

        lax.fori_loop(0, n_zero, zbody, 0)
        plsc.subcore_barrier()

        sbufs = (src_v, src_v2)
        dbufs = (dst_v, dst_v2)
        pltpu.sync_copy(idx_hbm.at[2 * r, c, s, 0], src_v)
        pltpu.sync_copy(idx_hbm.at[2 * r + 1, c, s, 0], dst_v)
        for q in range(NQ):
            sv = sbufs[q % 2]
            dv = dbufs[q % 2]
            if q + 1 < NQ:
                pltpu.async_copy(
                    idx_hbm.at[2 * r, c, s, q + 1], sbufs[(q + 1) % 2], sem_i)
                pltpu.async_copy(
                    idx_hbm.at[2 * r + 1, c, s, q + 1], dbufs[(q + 1) % 2],
                    sem_i)

            def startg(w, buf, sem, g=g, sv=sv):
                pltpu.async_copy(g.at[sv.at[w]], buf, sem)

            def waitg(w, buf, sem, g=g, sv=sv):
                pltpu.make_async_copy(g.at[sv.at[w]], buf, sem).wait()

            def starts(w, buf, sem, dv=dv):
                pltpu.async_copy(buf, agg_sh.at[dv.at[w]], sem, add=True)

            def waits(w, buf, sem, dv=dv):
                pltpu.make_async_copy(buf, agg_sh.at[dv.at[w]], sem).wait()

            startg(0, rows_a, sem_ga)
            startg(1, rows_b, sem_gb)
            waitg(0, rows_a, sem_ga)
            starts(0, rows_a, sem_sa)

            def pair(m, carry2, startg=startg, waitg=waitg, starts=starts,
                     waits=waits):
                w = 2 * m
                waits(w - 2, rows_a, sem_sa)
                startg(w, rows_a, sem_ga)
                waitg(w - 1, rows_b, sem_gb)
                starts(w - 1, rows_b, sem_sb)
                waits(w - 1, rows_b, sem_sb)
                startg(w + 1, rows_b, sem_gb)
                waitg(w, rows_a, sem_ga)
                starts(w, rows_a, sem_sa)
                return carry2

            lax.fori_loop(1, (NJ - 1) // 2, pair, 0)
            waits(NJ - 3, rows_a, sem_sa)
            startg(NJ - 1, rows_a, sem_ga)
            waitg(NJ - 2, rows_b, sem_gb)
            starts(NJ - 2, rows_b, sem_sb)
            waitg(NJ - 1, rows_a, sem_ga)
            starts(NJ - 1, rows_a, sem_sa)
            waits(NJ - 2, rows_b, sem_sb)
            waits(NJ - 1, rows_a, sem_sa)
            if q + 1 < NQ:
                pltpu.make_async_copy(
                    idx_hbm.at[2 * r, c, s, q + 1], sbufs[(q + 1) % 2],
                    sem_i).wait()
                pltpu.make_async_copy(
                    idx_hbm.at[2 * r + 1, c, s, q + 1], dbufs[(q + 1) % 2],
                    sem_i).wait()
        plsc.subcore_barrier()

        @pl.when(s < NS - 1)
        def _(r=r):
            pltpu.sync_copy(agg_sh.at[pl.ds(base, FROWS), :],
                            ps[r].at[c, pl.ds(base, FROWS), :])

        @pl.when(s == NS - 1)
        def _(r=r):
            pltpu.sync_copy(agg_sh.at[pl.ds(base, LROWS), :],
                            ps[r].at[c, pl.ds(base, LROWS), :])

        plsc.subcore_barrier()


R_BLK = 1000
GRID = N // R_BLK


def _tc_prep_body(x_ref, deg_ref, norms_ref, g0_ref, g1_ref, g2_ref):
    x = x_ref[...]
    dd = deg_ref[...]
    cols = []
    for k in range(6):
        d = dd[k, 0] + dd[k, 1]
        cols.append(jnp.where(d > 0, lax.rsqrt(jnp.maximum(d, 1e-30)), 0.0))
    pad = jnp.zeros_like(cols[0])
    norms_ref[...] = jnp.concatenate(cols + [pad, pad], axis=1)
    g_refs = (g0_ref, g1_ref, g2_ref)
    for r in range(3):
        g_refs[r][...] = x * cols[2 * r]


def _tc_prep(x, degp):
    return pl.pallas_call(
        _tc_prep_body,
        grid=(GRID,),
        in_specs=[
            pl.BlockSpec((R_BLK, D), lambda i: (i, 0)),
            pl.BlockSpec((6, NC, R_BLK, 1), lambda i: (0, 0, i, 0)),
        ],
        out_specs=[
            pl.BlockSpec((R_BLK, DEGW), lambda i: (i, 0)),
            pl.BlockSpec((R_BLK, D), lambda i: (i, 0)),
            pl.BlockSpec((R_BLK, D), lambda i: (i, 0)),
            pl.BlockSpec((R_BLK, D), lambda i: (i, 0)),
        ],
        out_shape=[
            jax.ShapeDtypeStruct((N, DEGW), jnp.float32),
            jax.ShapeDtypeStruct((N, D), jnp.float32),
            jax.ShapeDtypeStruct((N, D), jnp.float32),
            jax.ShapeDtypeStruct((N, D), jnp.float32),
        ],
    )(x, degp)


def _tc_layer_body(p0_ref, p1_ref, p2_ref, norms_ref, w_ref, b_ref, *out_refs,
                   rescale):
    norms = norms_ref[...]
    mats = []
    for r in range(3):
        pr = (p0_ref, p1_ref, p2_ref)[r][...]
        agg = pr[0] + pr[1]
        mats.append(agg * norms[:, 2 * r + 1:2 * r + 2])
    cat = jnp.concatenate(mats, axis=1)
    t = jnp.dot(cat, w_ref[...], preferred_element_type=jnp.float32)
    t = t + b_ref[...]
    if rescale:
        h = jnp.maximum(t, 0.0)
        for r in range(3):
            out_refs[r][...] = h * norms[:, 2 * r:2 * r + 1]
    else:
        out_refs[0][...] = t


def _tc_layer(p0, p1, p2, norms, wcat, bsum, rescale):
    n_out = 3 if rescale else 1
    return pl.pallas_call(
        functools.partial(_tc_layer_body, rescale=rescale),
        grid=(GRID,),
        in_specs=[
            pl.BlockSpec((NC, R_BLK, D), lambda i: (0, i, 0)),
            pl.BlockSpec((NC, R_BLK, D), lambda i: (0, i, 0)),
            pl.BlockSpec((NC, R_BLK, D), lambda i: (0, i, 0)),
            pl.BlockSpec((R_BLK, DEGW), lambda i: (i, 0)),
            pl.BlockSpec((3 * D, D), lambda i: (0, 0)),
            pl.BlockSpec((1, D), lambda i: (0, 0)),
        ],
        out_specs=[pl.BlockSpec((R_BLK, D), lambda i: (i, 0))] * n_out,
        out_shape=[jax.ShapeDtypeStruct((N, D), jnp.float32)] * n_out,
    )(p0, p1, p2, norms, wcat, bsum)


def kernel(x, edge_index_b, edge_index_r, edge_index_j,
           W1_b, W1_r, W1_j, b1_b, b1_r, b1_j,
           W2_b, W2_r, W2_j, b2_b, b2_r, b2_j):
    idx_stack = jnp.stack([
        edge_index_b[0], edge_index_b[1],
        edge_index_r[0], edge_index_r[1],
        edge_index_j[0], edge_index_j[1],
    ]).astype(jnp.int32)
    idx_all = idx_stack.reshape(6, NC, NS, NQ, NJ, CH)

    zeros_agg = jnp.zeros((ZCH, D), jnp.float32)
    zeros_hist = jnp.zeros((N_PAD,), jnp.float32)

    degp = _sc_degrees(idx_all, zeros_hist)
    degt = degp[:, :, :N, None]
    norms, g10, g11, g12 = _tc_prep(x, degt)

    p10, p11, p12 = _sc_aggregate(idx_all, g10, g11, g12, zeros_agg)
    w1cat = jnp.concatenate([W1_b, W1_r, W1_j], axis=0)
    b1sum = (b1_b + b1_r + b1_j)[None, :]
    g20, g21, g22 = _tc_layer(p10, p11, p12, norms, w1cat, b1sum, True)

    p20, p21, p22 = _sc_aggregate(idx_all, g20, g21, g22, zeros_agg)
    w2cat = jnp.concatenate([W2_b, W2_r, W2_j], axis=0)
    b2sum = (b2_b + b2_r + b2_j)[None, :]
    (out,) = _tc_layer(p20, p21, p22, norms, w2cat, b2sum, False)
    return out


# speedup vs baseline: 1.0674x; 1.0048x over previous
import functools

import jax
import jax.numpy as jnp
from jax import lax
from jax.experimental import pallas as pl
from jax.experimental.pallas import tpu as pltpu
from jax.experimental.pallas import tpu_sc as plsc

N = 10000
E = 320000
D = 128
NC = 2
NS = 16
CH = 80
NQ = 5
NJ = 25
NCHUNK = NQ * NJ
N_PAD = 10240
FROWS = N_PAD // NS
LROWS = N - (NS - 1) * FROWS
ZCH = 40
DEGW = 8

_mesh = plsc.VectorSubcoreMesh(
    core_axis_name="c", subcore_axis_name="s", num_cores=NC, num_subcores=NS)


@functools.partial(
    pl.kernel,
    out_type=jax.ShapeDtypeStruct((6, NC, N_PAD), jnp.float32),
    mesh=_mesh,
    scratch_types=[
        pltpu.VMEM((NJ, CH), jnp.int32),
        pltpu.VMEM((NJ, CH), jnp.int32),
        pltpu.VMEM((N_PAD,), jnp.float32),
        pltpu.VMEM((FROWS,), jnp.float32),
        pltpu.VMEM((FROWS,), jnp.float32),
        pltpu.VMEM_SHARED((NS, N_PAD), jnp.float32),
        pltpu.SemaphoreType.DMA,
    ],
    compiler_params=pltpu.CompilerParams(needs_layout_passes=False),
)
def _sc_degrees(idx_hbm, zeros_hbm, deg_hbm,
                idx_v, idx_v2, hist_v, acc_v, tmp_v, sh, sem_i):
    c = lax.axis_index("c")
    s = lax.axis_index("s")
    zeros16 = jnp.zeros((16,), jnp.float32)
    ones16 = jnp.ones((16,), jnp.float32)
    lane_base = pl.multiple_of(s * FROWS, 8)
    ibufs = (idx_v, idx_v2)
    for k in range(6):
        pltpu.sync_copy(zeros_hbm, hist_v)
        pltpu.sync_copy(idx_hbm.at[k, c, s, 0], idx_v)
        for q in range(NQ):
            iv = ibufs[q % 2]
            if q + 1 < NQ:
                pltpu.async_copy(
                    idx_hbm.at[k, c, s, q + 1], ibufs[(q + 1) % 2], sem_i)

            def jbody(j, carry2, iv=iv):
                for t in range(CH // 16):
                    v = iv[j, pl.ds(t * 16, 16)]
                    plsc.addupdate_scatter(hist_v, [v], ones16)
                return carry2

            lax.fori_loop(0, NJ, jbody, 0)
            if q + 1 < NQ:
                pltpu.make_async_copy(
                    idx_hbm.at[k, c, s, q + 1], ibufs[(q + 1) % 2],
                    sem_i).wait()
        pltpu.sync_copy(hist_v, sh.at[s])
        plsc.subcore_barrier()

        def zb2(i, carry):
            acc_v[pl.ds(i * 16, 16)] = zeros16
            return carry

        lax.fori_loop(0, FROWS // 16, zb2, 0)
        for i in range(NS):
            pltpu.sync_copy(sh.at[i, pl.ds(lane_base, FROWS)], tmp_v)

            def ab(m, carry):
                sl = pl.ds(m * 16, 16)
                acc_v[sl] = acc_v[sl] + tmp_v[sl]
                return carry

            lax.fori_loop(0, FROWS // 16, ab, 0)
        pltpu.sync_copy(acc_v, deg_hbm.at[k, c, pl.ds(lane_base, FROWS)])
        plsc.subcore_barrier()


_PART_T = jax.ShapeDtypeStruct((NC, N, D), jnp.float32)


@functools.partial(
    pl.kernel,
    out_type=(_PART_T, _PART_T, _PART_T),
    mesh=_mesh,
    scratch_types=[
        pltpu.VMEM((NJ, CH), jnp.int32),
        pltpu.VMEM((NJ, CH), jnp.int32),
        pltpu.VMEM((NJ, CH), jnp.int32),
        pltpu.VMEM((NJ, CH), jnp.int32),
        pltpu.VMEM((CH, D), jnp.float32),
        pltpu.VMEM((CH, D), jnp.float32),
        pltpu.VMEM((ZCH, D), jnp.float32),
        pltpu.VMEM_SHARED((N_PAD, D), jnp.float32),
        pltpu.SemaphoreType.DMA,
        pltpu.SemaphoreType.DMA,
        pltpu.SemaphoreType.DMA,
        pltpu.SemaphoreType.DMA,
        pltpu.SemaphoreType.DMA,
        pltpu.SemaphoreType.DMA,
    ],
)
def _sc_aggregate(idx_hbm, g0_hbm, g1_hbm, g2_hbm, zeros_hbm,
                  p0_hbm, p1_hbm, p2_hbm,
                  src_v, src_v2, dst_v, dst_v2, rows_a, rows_b, zero_v, agg_sh,
                  sem_ga, sem_gb, sem_sa, sem_sb, sem_i, sem_z):
    c = lax.axis_index("c")
    s = lax.axis_index("s")
    base = pl.multiple_of(s * FROWS, 8)
    n_zero = jnp.where(s < NS - 1, FROWS // ZCH, LROWS // ZCH)
    pltpu.sync_copy(zeros_hbm, zero_v)
    gs = (g0_hbm, g1_hbm, g2_hbm)
    ps = (p0_hbm, p1_hbm, p2_hbm)
    for r in range(3):
        g = gs[r]

        def zbody(z, carry):
            off = pl.multiple_of(base + z * ZCH, 8)
            pltpu.async_copy(zero_v, agg_sh.at[pl.ds(off, ZCH), :], sem_z)
            return carry

        lax.fori_loop(0, n_zero, zbody, 0)

        def zwait(z, carry):
            off = pl.multiple_of(base + z * ZCH, 8)
            pltpu.make_async_copy(
                zero_v, agg_sh.at[pl.ds(off, ZCH), :], sem_z).wait()
            return carry

        lax.fori_loop(0, n_zero, zwait, 0)
        plsc.subcore_barrier()

        sbufs = (src_v, src_v2)
        dbufs = (dst_v, dst_v2)
        pltpu.sync_copy(idx_hbm.at[2 * r, c, s, 0], src_v)
        pltpu.sync_copy(idx_hbm.at[2 * r + 1, c, s, 0], dst_v)
        for q in range(NQ):
            sv = sbufs[q % 2]
            dv = dbufs[q % 2]
            if q + 1 < NQ:
                pltpu.async_copy(
                    idx_hbm.at[2 * r, c, s, q + 1], sbufs[(q + 1) % 2], sem_i)
                pltpu.async_copy(
                    idx_hbm.at[2 * r + 1, c, s, q + 1], dbufs[(q + 1) % 2],
                    sem_i)

            def startg(w, buf, sem, g=g, sv=sv):
                pltpu.async_copy(g.at[sv.at[w]], buf, sem)

            def waitg(w, buf, sem, g=g, sv=sv):
                pltpu.make_async_copy(g.at[sv.at[w]], buf, sem).wait()

            def starts(w, buf, sem, dv=dv):
                pltpu.async_copy(buf, agg_sh.at[dv.at[w]], sem, add=True)

            def waits(w, buf, sem, dv=dv):
                pltpu.make_async_copy(buf, agg_sh.at[dv.at[w]], sem).wait()

            startg(0, rows_a, sem_ga)
            startg(1, rows_b, sem_gb)
            waitg(0, rows_a, sem_ga)
            starts(0, rows_a, sem_sa)

            def pair(m, carry2, startg=startg, waitg=waitg, starts=starts,
                     waits=waits):
                w = 2 * m
                waits(w - 2, rows_a, sem_sa)
                startg(w, rows_a, sem_ga)
                waitg(w - 1, rows_b, sem_gb)
                starts(w - 1, rows_b, sem_sb)
                waits(w - 1, rows_b, sem_sb)
                startg(w + 1, rows_b, sem_gb)
                waitg(w, rows_a, sem_ga)
                starts(w, rows_a, sem_sa)
                return carry2

            lax.fori_loop(1, (NJ - 1) // 2, pair, 0)
            waits(NJ - 3, rows_a, sem_sa)
            startg(NJ - 1, rows_a, sem_ga)
            waitg(NJ - 2, rows_b, sem_gb)
            starts(NJ - 2, rows_b, sem_sb)
            waitg(NJ - 1, rows_a, sem_ga)
            starts(NJ - 1, rows_a, sem_sa)
            waits(NJ - 2, rows_b, sem_sb)
            waits(NJ - 1, rows_a, sem_sa)
            if q + 1 < NQ:
                pltpu.make_async_copy(
                    idx_hbm.at[2 * r, c, s, q + 1], sbufs[(q + 1) % 2],
                    sem_i).wait()
                pltpu.make_async_copy(
                    idx_hbm.at[2 * r + 1, c, s, q + 1], dbufs[(q + 1) % 2],
                    sem_i).wait()
        plsc.subcore_barrier()

        @pl.when(s < NS - 1)
        def _(r=r):
            pltpu.sync_copy(agg_sh.at[pl.ds(base, FROWS), :],
                            ps[r].at[c, pl.ds(base, FROWS), :])

        @pl.when(s == NS - 1)
        def _(r=r):
            pltpu.sync_copy(agg_sh.at[pl.ds(base, LROWS), :],
                            ps[r].at[c, pl.ds(base, LROWS), :])

        plsc.subcore_barrier()


R_BLK = 1000
GRID = N // R_BLK


def _tc_prep_body(x_ref, deg_ref, norms_ref, g0_ref, g1_ref, g2_ref):
    x = x_ref[...]
    dd = deg_ref[...]
    cols = []
    for k in range(6):
        d = dd[k, 0] + dd[k, 1]
        cols.append(jnp.where(d > 0, lax.rsqrt(jnp.maximum(d, 1e-30)), 0.0))
    pad = jnp.zeros_like(cols[0])
    norms_ref[...] = jnp.concatenate(cols + [pad, pad], axis=1)
    g_refs = (g0_ref, g1_ref, g2_ref)
    for r in range(3):
        g_refs[r][...] = x * cols[2 * r]


def _tc_prep(x, degp):
    return pl.pallas_call(
        _tc_prep_body,
        grid=(GRID,),
        in_specs=[
            pl.BlockSpec((R_BLK, D), lambda i: (i, 0)),
            pl.BlockSpec((6, NC, R_BLK, 1), lambda i: (0, 0, i, 0)),
        ],
        out_specs=[
            pl.BlockSpec((R_BLK, DEGW), lambda i: (i, 0)),
            pl.BlockSpec((R_BLK, D), lambda i: (i, 0)),
            pl.BlockSpec((R_BLK, D), lambda i: (i, 0)),
            pl.BlockSpec((R_BLK, D), lambda i: (i, 0)),
        ],
        out_shape=[
            jax.ShapeDtypeStruct((N, DEGW), jnp.float32),
            jax.ShapeDtypeStruct((N, D), jnp.float32),
            jax.ShapeDtypeStruct((N, D), jnp.float32),
            jax.ShapeDtypeStruct((N, D), jnp.float32),
        ],
    )(x, degp)


def _tc_layer_body(p0_ref, p1_ref, p2_ref, norms_ref, w_ref, b_ref, *out_refs,
                   rescale):
    norms = norms_ref[...]
    mats = []
    for r in range(3):
        pr = (p0_ref, p1_ref, p2_ref)[r][...]
        agg = pr[0] + pr[1]
        mats.append(agg * norms[:, 2 * r + 1:2 * r + 2])
    cat = jnp.concatenate(mats, axis=1)
    t = jnp.dot(cat, w_ref[...], preferred_element_type=jnp.float32)
    t = t + b_ref[...]
    if rescale:
        h = jnp.maximum(t, 0.0)
        for r in range(3):
            out_refs[r][...] = h * norms[:, 2 * r:2 * r + 1]
    else:
        out_refs[0][...] = t


def _tc_layer(p0, p1, p2, norms, wcat, bsum, rescale):
    n_out = 3 if rescale else 1
    return pl.pallas_call(
        functools.partial(_tc_layer_body, rescale=rescale),
        grid=(GRID,),
        in_specs=[
            pl.BlockSpec((NC, R_BLK, D), lambda i: (0, i, 0)),
            pl.BlockSpec((NC, R_BLK, D), lambda i: (0, i, 0)),
            pl.BlockSpec((NC, R_BLK, D), lambda i: (0, i, 0)),
            pl.BlockSpec((R_BLK, DEGW), lambda i: (i, 0)),
            pl.BlockSpec((3 * D, D), lambda i: (0, 0)),
            pl.BlockSpec((1, D), lambda i: (0, 0)),
        ],
        out_specs=[pl.BlockSpec((R_BLK, D), lambda i: (i, 0))] * n_out,
        out_shape=[jax.ShapeDtypeStruct((N, D), jnp.float32)] * n_out,
    )(p0, p1, p2, norms, wcat, bsum)


def kernel(x, edge_index_b, edge_index_r, edge_index_j,
           W1_b, W1_r, W1_j, b1_b, b1_r, b1_j,
           W2_b, W2_r, W2_j, b2_b, b2_r, b2_j):
    idx_stack = jnp.stack([
        edge_index_b[0], edge_index_b[1],
        edge_index_r[0], edge_index_r[1],
        edge_index_j[0], edge_index_j[1],
    ]).astype(jnp.int32)
    idx_all = idx_stack.reshape(6, NC, NS, NQ, NJ, CH)

    zeros_agg = jnp.zeros((ZCH, D), jnp.float32)
    zeros_hist = jnp.zeros((N_PAD,), jnp.float32)

    degp = _sc_degrees(idx_all, zeros_hist)
    degt = degp[:, :, :N, None]
    norms, g10, g11, g12 = _tc_prep(x, degt)

    p10, p11, p12 = _sc_aggregate(idx_all, g10, g11, g12, zeros_agg)
    w1cat = jnp.concatenate([W1_b, W1_r, W1_j], axis=0)
    b1sum = (b1_b + b1_r + b1_j)[None, :]
    g20, g21, g22 = _tc_layer(p10, p11, p12, norms, w1cat, b1sum, True)

    p20, p21, p22 = _sc_aggregate(idx_all, g20, g21, g22, zeros_agg)
    w2cat = jnp.concatenate([W2_b, W2_r, W2_j], axis=0)
    b2sum = (b2_b + b2_r + b2_j)[None, :]
    (out,) = _tc_layer(p20, p21, p22, norms, w2cat, b2sum, False)
    return out
